# baseline (speedup 1.0000x reference)
import functools

import jax
import jax.numpy as jnp
from jax import lax
from jax.experimental import pallas as pl
from jax.experimental.pallas import tpu as pltpu
from jax.experimental.pallas import tpu_sc as plsc

_mesh = plsc.VectorSubcoreMesh(core_axis_name="c", subcore_axis_name="s")


@functools.partial(
    pl.kernel,
    mesh=_mesh,
    out_type=jax.ShapeDtypeStruct((1,), jnp.float32),
    scratch_types=[
        pltpu.VMEM((16,), jnp.int32),
        pltpu.VMEM((30,), jnp.float32),
        pltpu.VMEM((16,), jnp.float32),
        pltpu.VMEM((16,), jnp.float32),
    ],
    compiler_params=pltpu.CompilerParams(needs_layout_passes=False),
)
def _embed(i_hbm, w_hbm, b_hbm, out_hbm, i_v, w_v, b_v, o_v):
    wid = lax.axis_index("s") * 2 + lax.axis_index("c")

    @pl.when(wid == 0)
    def _():
        pltpu.sync_copy(i_hbm, i_v.at[pl.ds(0, 1)])
        pltpu.sync_copy(w_hbm, w_v)
        pltpu.sync_copy(b_hbm, b_v.at[pl.ds(0, 1)])
        idx = i_v[...][0]
        idxs = jnp.broadcast_to(idx, (16,))
        gathered = plsc.load_gather(w_v, [idxs])
        o_v[...] = gathered + b_v[...]
        pltpu.sync_copy(o_v.at[pl.ds(0, 1)], out_hbm)


def kernel(i, W, b):
    i_arr = jnp.asarray(i, dtype=jnp.int32).reshape((1,))
    w_flat = W.reshape((30,))
    return _embed(i_arr, w_flat, b)


# speedup vs baseline: 1.1567x; 1.1567x over previous
import functools

import jax
import jax.numpy as jnp
from jax.experimental import pallas as pl
from jax.experimental.pallas import tpu as pltpu
from jax.experimental.pallas import tpu_sc as plsc

_mesh = plsc.ScalarSubcoreMesh(axis_name="c", num_cores=1)


@functools.partial(
    pl.kernel,
    mesh=_mesh,
    out_type=jax.ShapeDtypeStruct((1,), jnp.float32),
    scratch_types=[
        pltpu.SMEM((1,), jnp.int32),
        pltpu.SMEM((30,), jnp.float32),
        pltpu.SMEM((1,), jnp.float32),
        pltpu.SMEM((1,), jnp.float32),
    ],
    compiler_params=pltpu.CompilerParams(needs_layout_passes=False),
)
def _embed(i_hbm, w_hbm, b_hbm, out_hbm, i_s, w_s, b_s, o_s):
    pltpu.sync_copy(i_hbm, i_s)
    pltpu.sync_copy(w_hbm, w_s)
    pltpu.sync_copy(b_hbm, b_s)
    o_s[0] = w_s[i_s[0]] + b_s[0]
    pltpu.sync_copy(o_s, out_hbm)


def kernel(i, W, b):
    i_arr = jnp.asarray(i, dtype=jnp.int32).reshape((1,))
    w_flat = W.reshape((30,))
    return _embed(i_arr, w_flat, b)


# speedup vs baseline: 1.2359x; 1.0684x over previous
import functools

import jax
import jax.numpy as jnp
from jax.experimental import pallas as pl
from jax.experimental.pallas import tpu as pltpu
from jax.experimental.pallas import tpu_sc as plsc

_mesh = plsc.ScalarSubcoreMesh(axis_name="c", num_cores=1)


@functools.partial(
    pl.kernel,
    mesh=_mesh,
    out_type=jax.ShapeDtypeStruct((1,), jnp.float32),
    scratch_types=[
        pltpu.SMEM((1,), jnp.int32),
        pltpu.SMEM((30,), jnp.float32),
        pltpu.SMEM((1,), jnp.float32),
        pltpu.SMEM((1,), jnp.float32),
        pltpu.SemaphoreType.DMA,
    ],
    compiler_params=pltpu.CompilerParams(needs_layout_passes=False),
)
def _embed(i_hbm, w_hbm, b_hbm, out_hbm, i_s, w_s, b_s, o_s, sem):
    c1 = pltpu.make_async_copy(i_hbm, i_s, sem)
    c2 = pltpu.make_async_copy(w_hbm, w_s, sem)
    c3 = pltpu.make_async_copy(b_hbm, b_s, sem)
    c1.start()
    c2.start()
    c3.start()
    c1.wait()
    c2.wait()
    c3.wait()
    o_s[0] = w_s[i_s[0]] + b_s[0]
    pltpu.sync_copy(o_s, out_hbm)


def kernel(i, W, b):
    i_arr = jnp.asarray(i, dtype=jnp.int32).reshape((1,))
    w_flat = W.reshape((30,))
    return _embed(i_arr, w_flat, b)
